# baseline (speedup 1.0000x reference)
import jax
import jax.numpy as jnp
from jax.experimental import pallas as pl
from jax.experimental.pallas import tpu as pltpu

N = 10000
D = 256
DH = 128
NC = 8
CHUNK = 1000
NCH = N // CHUNK
ALPHA = 0.5
EPS = 1e-5


def _dot_t(a, b):
    return jax.lax.dot_general(
        a, b, (((1,), (1,)), ((), ())), preferred_element_type=jnp.float32)


def _gnn_kernel(x_ref, comm_ref,
                wv0, wo0, wp0, g0, be0,
                wv1, wo1, wp1, g1, be1,
                wv2, wo2, wp2, g2, be2,
                cemb, wvc, woc, bvc, boc,
                w1, b1, w2, b2,
                out_ref, h_ref):
    def fold(wp, wo, wv):
        return jnp.dot(wp[...], jnp.dot(wo[...], wv[...],
                                        preferred_element_type=jnp.float32,
                                        precision=jax.lax.Precision.HIGHEST),
                       preferred_element_type=jnp.float32,
                       precision=jax.lax.Precision.HIGHEST)

    a0 = fold(wp0, wo0, wv0)
    a1 = fold(wp1, wo1, wv1)
    a2 = fold(wp2, wo2, wv2)

    def dot_t_hi(a, b):
        return jax.lax.dot_general(
            a, b, (((1,), (1,)), ((), ())), preferred_element_type=jnp.float32,
            precision=jax.lax.Precision.HIGHEST)

    v8 = dot_t_hi(cemb[...], wvc[...]) + bvc[...]
    table = dot_t_hi(v8, woc[...]) + boc[...]

    zero = jnp.zeros((1, D), jnp.float32)

    def stats_to_affine(s, q, gamma, beta):
        mu = s * (1.0 / N)
        var = q * (1.0 / N) - mu * mu
        inv = jax.lax.rsqrt(var + EPS)
        scale = gamma[...] * inv
        shift = beta[...] - mu * scale
        return scale, shift

    def att_chunk(i):
        cc = comm_ref[pl.ds(i * CHUNK, CHUNK), :]
        oh = (cc == jax.lax.broadcasted_iota(jnp.int32, (CHUNK, NC), 1))
        return jax.lax.dot_general(
            oh.astype(jnp.float32), table, (((1,), (0,)), ((), ())),
            preferred_element_type=jnp.float32)

    def pass_a(i, carry):
        s, q = carry
        y = _dot_t(x_ref[pl.ds(i * CHUNK, CHUNK), :], a0)
        out_ref[pl.ds(i * CHUNK, CHUNK), :] = y
        return s + jnp.sum(y, 0, keepdims=True), q + jnp.sum(y * y, 0, keepdims=True)

    s, q = jax.lax.fori_loop(0, NCH, pass_a, (zero, zero))
    scale0, shift0 = stats_to_affine(s, q, g0, be0)

    def pass_b(i, carry):
        s, q = carry
        y = out_ref[pl.ds(i * CHUNK, CHUNK), :]
        h1 = y * scale0 + shift0
        gmix = ALPHA * att_chunk(i) + (1.0 - ALPHA) * h1
        h_ref[pl.ds(i * CHUNK, CHUNK), :] = gmix
        y1 = _dot_t(gmix, a1)
        out_ref[pl.ds(i * CHUNK, CHUNK), :] = y1
        return s + jnp.sum(y1, 0, keepdims=True), q + jnp.sum(y1 * y1, 0, keepdims=True)

    s, q = jax.lax.fori_loop(0, NCH, pass_b, (zero, zero))
    scale1, shift1 = stats_to_affine(s, q, g1, be1)

    def pass_c(i, carry):
        s, q = carry
        y = out_ref[pl.ds(i * CHUNK, CHUNK), :]
        h2 = y * scale1 + shift1 + h_ref[pl.ds(i * CHUNK, CHUNK), :]
        gmix = ALPHA * att_chunk(i) + (1.0 - ALPHA) * h2
        h_ref[pl.ds(i * CHUNK, CHUNK), :] = gmix
        y2 = _dot_t(gmix, a2)
        out_ref[pl.ds(i * CHUNK, CHUNK), :] = y2
        return s + jnp.sum(y2, 0, keepdims=True), q + jnp.sum(y2 * y2, 0, keepdims=True)

    s, q = jax.lax.fori_loop(0, NCH, pass_c, (zero, zero))
    scale2, shift2 = stats_to_affine(s, q, g2, be2)

    def pass_d(i, _):
        y = out_ref[pl.ds(i * CHUNK, CHUNK), :]
        h3 = y * scale2 + shift2 + h_ref[pl.ds(i * CHUNK, CHUNK), :]
        hid = jax.nn.relu(_dot_t(h3, w1[...]) + b1[...])
        out_ref[pl.ds(i * CHUNK, CHUNK), :] = _dot_t(hid, w2[...]) + b2[...]
        return 0

    jax.lax.fori_loop(0, NCH, pass_d, 0)


def kernel(x, edge_index, communities, params):
    del edge_index
    p0, p1, p2 = params['layer0'], params['layer1'], params['layer2']
    ca = params['comm_attn']

    def row(v):
        return v.reshape(1, -1)

    args = (
        x, communities.reshape(N, 1).astype(jnp.int32),
        p0['Wv'], p0['Wo'], p0['Wp'], row(p0['gamma']), row(p0['beta']),
        p1['Wv'], p1['Wo'], p1['Wp'], row(p1['gamma']), row(p1['beta']),
        p2['Wv'], p2['Wo'], p2['Wp'], row(p2['gamma']), row(p2['beta']),
        params['comm_emb'], ca['Wv'], ca['Wo'], row(ca['bv']), row(ca['bo']),
        params['pred_W1'], row(params['pred_b1']),
        params['pred_W2'], row(params['pred_b2']),
    )

    return pl.pallas_call(
        _gnn_kernel,
        out_shape=jax.ShapeDtypeStruct((N, D), jnp.float32),
        scratch_shapes=[pltpu.VMEM((N, D), jnp.float32)],
    )(*args)


# speedup vs baseline: 1.1896x; 1.1896x over previous
import jax
import jax.numpy as jnp
from jax.experimental import pallas as pl
from jax.experimental.pallas import tpu as pltpu

N = 10000
D = 256
DH = 128
NC = 8
CHUNK = 2000
NCH = N // CHUNK
ALPHA = 0.5
EPS = 1e-5


def _dot_t(a, b):
    return jax.lax.dot_general(
        a, b, (((1,), (1,)), ((), ())), preferred_element_type=jnp.float32)


def _gnn_kernel(x_ref, comm_ref,
                wv0, wo0, wp0, g0, be0,
                wv1, wo1, wp1, g1, be1,
                wv2, wo2, wp2, g2, be2,
                cemb, wvc, woc, bvc, boc,
                w1, b1, w2, b2,
                out_ref, h_ref):
    def fold(wp, wo, wv):
        return jnp.dot(wp[...], jnp.dot(wo[...], wv[...],
                                        preferred_element_type=jnp.float32),
                       preferred_element_type=jnp.float32)

    a0 = fold(wp0, wo0, wv0)
    a1 = fold(wp1, wo1, wv1)
    a2 = fold(wp2, wo2, wv2)

    v8 = _dot_t(cemb[...], wvc[...]) + bvc[...]
    table = _dot_t(v8, woc[...]) + boc[...]

    zero = jnp.zeros((1, D), jnp.float32)

    def stats_to_affine(s, q, gamma, beta):
        mu = s * (1.0 / N)
        var = q * (1.0 / N) - mu * mu
        inv = jax.lax.rsqrt(var + EPS)
        scale = gamma[...] * inv
        shift = beta[...] - mu * scale
        return scale, shift

    def att_chunk(i):
        cc = comm_ref[pl.ds(i * CHUNK, CHUNK), :]
        oh = (cc == jax.lax.broadcasted_iota(jnp.int32, (CHUNK, NC), 1))
        return jax.lax.dot_general(
            oh.astype(jnp.float32), table, (((1,), (0,)), ((), ())),
            preferred_element_type=jnp.float32)

    def pass_a(i, carry):
        s, q = carry
        y = _dot_t(x_ref[pl.ds(i * CHUNK, CHUNK), :], a0)
        out_ref[pl.ds(i * CHUNK, CHUNK), :] = y
        return s + jnp.sum(y, 0, keepdims=True), q + jnp.sum(y * y, 0, keepdims=True)

    s, q = jax.lax.fori_loop(0, NCH, pass_a, (zero, zero))
    scale0, shift0 = stats_to_affine(s, q, g0, be0)

    def pass_b(i, carry):
        s, q = carry
        y = out_ref[pl.ds(i * CHUNK, CHUNK), :]
        h1 = y * scale0 + shift0
        gmix = ALPHA * att_chunk(i) + (1.0 - ALPHA) * h1
        h_ref[pl.ds(i * CHUNK, CHUNK), :] = gmix
        y1 = _dot_t(gmix, a1)
        out_ref[pl.ds(i * CHUNK, CHUNK), :] = y1
        return s + jnp.sum(y1, 0, keepdims=True), q + jnp.sum(y1 * y1, 0, keepdims=True)

    s, q = jax.lax.fori_loop(0, NCH, pass_b, (zero, zero))
    scale1, shift1 = stats_to_affine(s, q, g1, be1)

    def pass_c(i, carry):
        s, q = carry
        y = out_ref[pl.ds(i * CHUNK, CHUNK), :]
        h2 = y * scale1 + shift1 + h_ref[pl.ds(i * CHUNK, CHUNK), :]
        gmix = ALPHA * att_chunk(i) + (1.0 - ALPHA) * h2
        h_ref[pl.ds(i * CHUNK, CHUNK), :] = gmix
        y2 = _dot_t(gmix, a2)
        out_ref[pl.ds(i * CHUNK, CHUNK), :] = y2
        return s + jnp.sum(y2, 0, keepdims=True), q + jnp.sum(y2 * y2, 0, keepdims=True)

    s, q = jax.lax.fori_loop(0, NCH, pass_c, (zero, zero))
    scale2, shift2 = stats_to_affine(s, q, g2, be2)

    def pass_d(i, _):
        y = out_ref[pl.ds(i * CHUNK, CHUNK), :]
        h3 = y * scale2 + shift2 + h_ref[pl.ds(i * CHUNK, CHUNK), :]
        hid = jax.nn.relu(_dot_t(h3, w1[...]) + b1[...])
        out_ref[pl.ds(i * CHUNK, CHUNK), :] = _dot_t(hid, w2[...]) + b2[...]
        return 0

    jax.lax.fori_loop(0, NCH, pass_d, 0)


def kernel(x, edge_index, communities, params):
    del edge_index
    p0, p1, p2 = params['layer0'], params['layer1'], params['layer2']
    ca = params['comm_attn']

    def row(v):
        return v.reshape(1, -1)

    args = (
        x, communities.reshape(N, 1).astype(jnp.int32),
        p0['Wv'], p0['Wo'], p0['Wp'], row(p0['gamma']), row(p0['beta']),
        p1['Wv'], p1['Wo'], p1['Wp'], row(p1['gamma']), row(p1['beta']),
        p2['Wv'], p2['Wo'], p2['Wp'], row(p2['gamma']), row(p2['beta']),
        params['comm_emb'], ca['Wv'], ca['Wo'], row(ca['bv']), row(ca['bo']),
        params['pred_W1'], row(params['pred_b1']),
        params['pred_W2'], row(params['pred_b2']),
    )

    return pl.pallas_call(
        _gnn_kernel,
        out_shape=jax.ShapeDtypeStruct((N, D), jnp.float32),
        scratch_shapes=[pltpu.VMEM((N, D), jnp.float32)],
    )(*args)


# speedup vs baseline: 1.2683x; 1.0662x over previous
import jax
import jax.numpy as jnp
from jax.experimental import pallas as pl
from jax.experimental.pallas import tpu as pltpu

N = 10000
D = 256
DH = 128
NC = 8
CHUNK = 5000
NCH = N // CHUNK
ALPHA = 0.5
EPS = 1e-5


def _dot_t(a, b):
    return jax.lax.dot_general(
        a, b, (((1,), (1,)), ((), ())), preferred_element_type=jnp.float32)


def _gnn_kernel(x_ref, comm_ref,
                wv0, wo0, wp0, g0, be0,
                wv1, wo1, wp1, g1, be1,
                wv2, wo2, wp2, g2, be2,
                cemb, wvc, woc, bvc, boc,
                w1, b1, w2, b2,
                out_ref, h_ref):
    def fold(wp, wo, wv):
        return jnp.dot(wp[...], jnp.dot(wo[...], wv[...],
                                        preferred_element_type=jnp.float32),
                       preferred_element_type=jnp.float32)

    a0 = fold(wp0, wo0, wv0)
    a1 = fold(wp1, wo1, wv1)
    a2 = fold(wp2, wo2, wv2)

    v8 = _dot_t(cemb[...], wvc[...]) + bvc[...]
    table = _dot_t(v8, woc[...]) + boc[...]

    zero = jnp.zeros((1, D), jnp.float32)

    def stats_to_affine(s, q, gamma, beta):
        mu = s * (1.0 / N)
        var = q * (1.0 / N) - mu * mu
        inv = jax.lax.rsqrt(var + EPS)
        scale = gamma[...] * inv
        shift = beta[...] - mu * scale
        return scale, shift

    def att_chunk(i):
        cc = comm_ref[pl.ds(i * CHUNK, CHUNK), :]
        oh = (cc == jax.lax.broadcasted_iota(jnp.int32, (CHUNK, NC), 1))
        return jax.lax.dot_general(
            oh.astype(jnp.float32), table, (((1,), (0,)), ((), ())),
            preferred_element_type=jnp.float32)

    def pass_a(i, carry):
        s, q = carry
        y = _dot_t(x_ref[pl.ds(i * CHUNK, CHUNK), :], a0)
        out_ref[pl.ds(i * CHUNK, CHUNK), :] = y
        return s + jnp.sum(y, 0, keepdims=True), q + jnp.sum(y * y, 0, keepdims=True)

    s, q = jax.lax.fori_loop(0, NCH, pass_a, (zero, zero))
    scale0, shift0 = stats_to_affine(s, q, g0, be0)

    def pass_b(i, carry):
        s, q = carry
        y = out_ref[pl.ds(i * CHUNK, CHUNK), :]
        h1 = y * scale0 + shift0
        gmix = ALPHA * att_chunk(i) + (1.0 - ALPHA) * h1
        h_ref[pl.ds(i * CHUNK, CHUNK), :] = gmix
        y1 = _dot_t(gmix, a1)
        out_ref[pl.ds(i * CHUNK, CHUNK), :] = y1
        return s + jnp.sum(y1, 0, keepdims=True), q + jnp.sum(y1 * y1, 0, keepdims=True)

    s, q = jax.lax.fori_loop(0, NCH, pass_b, (zero, zero))
    scale1, shift1 = stats_to_affine(s, q, g1, be1)

    def pass_c(i, carry):
        s, q = carry
        y = out_ref[pl.ds(i * CHUNK, CHUNK), :]
        h2 = y * scale1 + shift1 + h_ref[pl.ds(i * CHUNK, CHUNK), :]
        gmix = ALPHA * att_chunk(i) + (1.0 - ALPHA) * h2
        h_ref[pl.ds(i * CHUNK, CHUNK), :] = gmix
        y2 = _dot_t(gmix, a2)
        out_ref[pl.ds(i * CHUNK, CHUNK), :] = y2
        return s + jnp.sum(y2, 0, keepdims=True), q + jnp.sum(y2 * y2, 0, keepdims=True)

    s, q = jax.lax.fori_loop(0, NCH, pass_c, (zero, zero))
    scale2, shift2 = stats_to_affine(s, q, g2, be2)

    def pass_d(i, _):
        y = out_ref[pl.ds(i * CHUNK, CHUNK), :]
        h3 = y * scale2 + shift2 + h_ref[pl.ds(i * CHUNK, CHUNK), :]
        hid = jax.nn.relu(_dot_t(h3, w1[...]) + b1[...])
        out_ref[pl.ds(i * CHUNK, CHUNK), :] = _dot_t(hid, w2[...]) + b2[...]
        return 0

    jax.lax.fori_loop(0, NCH, pass_d, 0)


def kernel(x, edge_index, communities, params):
    del edge_index
    p0, p1, p2 = params['layer0'], params['layer1'], params['layer2']
    ca = params['comm_attn']

    def row(v):
        return v.reshape(1, -1)

    args = (
        x, communities.reshape(N, 1).astype(jnp.int32),
        p0['Wv'], p0['Wo'], p0['Wp'], row(p0['gamma']), row(p0['beta']),
        p1['Wv'], p1['Wo'], p1['Wp'], row(p1['gamma']), row(p1['beta']),
        p2['Wv'], p2['Wo'], p2['Wp'], row(p2['gamma']), row(p2['beta']),
        params['comm_emb'], ca['Wv'], ca['Wo'], row(ca['bv']), row(ca['bo']),
        params['pred_W1'], row(params['pred_b1']),
        params['pred_W2'], row(params['pred_b2']),
    )

    return pl.pallas_call(
        _gnn_kernel,
        out_shape=jax.ShapeDtypeStruct((N, D), jnp.float32),
        scratch_shapes=[pltpu.VMEM((N, D), jnp.float32)],
    )(*args)


# speedup vs baseline: 1.4628x; 1.1533x over previous
import jax
import jax.numpy as jnp
from jax.experimental import pallas as pl
from jax.experimental.pallas import tpu as pltpu

N = 10000
D = 256
DH = 128
NC = 8
CHUNK = 10000
NCH = N // CHUNK
ALPHA = 0.5
EPS = 1e-5


def _dot_t(a, b):
    return jax.lax.dot_general(
        a, b, (((1,), (1,)), ((), ())), preferred_element_type=jnp.float32)


def _gnn_kernel(x_ref, comm_ref,
                wv0, wo0, wp0, g0, be0,
                wv1, wo1, wp1, g1, be1,
                wv2, wo2, wp2, g2, be2,
                cemb, wvc, woc, bvc, boc,
                w1, b1, w2, b2,
                out_ref, h_ref):
    def fold(wp, wo, wv):
        return jnp.dot(wp[...], jnp.dot(wo[...], wv[...],
                                        preferred_element_type=jnp.float32),
                       preferred_element_type=jnp.float32)

    a0 = fold(wp0, wo0, wv0)
    a1 = fold(wp1, wo1, wv1)
    a2 = fold(wp2, wo2, wv2)

    v8 = _dot_t(cemb[...], wvc[...]) + bvc[...]
    table = _dot_t(v8, woc[...]) + boc[...]

    zero = jnp.zeros((1, D), jnp.float32)

    def stats_to_affine(s, q, gamma, beta):
        mu = s * (1.0 / N)
        var = q * (1.0 / N) - mu * mu
        inv = jax.lax.rsqrt(var + EPS)
        scale = gamma[...] * inv
        shift = beta[...] - mu * scale
        return scale, shift

    def att_chunk(i):
        cc = comm_ref[pl.ds(i * CHUNK, CHUNK), :]
        oh = (cc == jax.lax.broadcasted_iota(jnp.int32, (CHUNK, NC), 1))
        return jax.lax.dot_general(
            oh.astype(jnp.float32), table, (((1,), (0,)), ((), ())),
            preferred_element_type=jnp.float32)

    def pass_a(i, carry):
        s, q = carry
        y = _dot_t(x_ref[pl.ds(i * CHUNK, CHUNK), :], a0)
        out_ref[pl.ds(i * CHUNK, CHUNK), :] = y
        return s + jnp.sum(y, 0, keepdims=True), q + jnp.sum(y * y, 0, keepdims=True)

    s, q = jax.lax.fori_loop(0, NCH, pass_a, (zero, zero))
    scale0, shift0 = stats_to_affine(s, q, g0, be0)

    def pass_b(i, carry):
        s, q = carry
        y = out_ref[pl.ds(i * CHUNK, CHUNK), :]
        h1 = y * scale0 + shift0
        gmix = ALPHA * att_chunk(i) + (1.0 - ALPHA) * h1
        h_ref[pl.ds(i * CHUNK, CHUNK), :] = gmix
        y1 = _dot_t(gmix, a1)
        out_ref[pl.ds(i * CHUNK, CHUNK), :] = y1
        return s + jnp.sum(y1, 0, keepdims=True), q + jnp.sum(y1 * y1, 0, keepdims=True)

    s, q = jax.lax.fori_loop(0, NCH, pass_b, (zero, zero))
    scale1, shift1 = stats_to_affine(s, q, g1, be1)

    def pass_c(i, carry):
        s, q = carry
        y = out_ref[pl.ds(i * CHUNK, CHUNK), :]
        h2 = y * scale1 + shift1 + h_ref[pl.ds(i * CHUNK, CHUNK), :]
        gmix = ALPHA * att_chunk(i) + (1.0 - ALPHA) * h2
        h_ref[pl.ds(i * CHUNK, CHUNK), :] = gmix
        y2 = _dot_t(gmix, a2)
        out_ref[pl.ds(i * CHUNK, CHUNK), :] = y2
        return s + jnp.sum(y2, 0, keepdims=True), q + jnp.sum(y2 * y2, 0, keepdims=True)

    s, q = jax.lax.fori_loop(0, NCH, pass_c, (zero, zero))
    scale2, shift2 = stats_to_affine(s, q, g2, be2)

    def pass_d(i, _):
        y = out_ref[pl.ds(i * CHUNK, CHUNK), :]
        h3 = y * scale2 + shift2 + h_ref[pl.ds(i * CHUNK, CHUNK), :]
        hid = jax.nn.relu(_dot_t(h3, w1[...]) + b1[...])
        out_ref[pl.ds(i * CHUNK, CHUNK), :] = _dot_t(hid, w2[...]) + b2[...]
        return 0

    jax.lax.fori_loop(0, NCH, pass_d, 0)


def kernel(x, edge_index, communities, params):
    del edge_index
    p0, p1, p2 = params['layer0'], params['layer1'], params['layer2']
    ca = params['comm_attn']

    def row(v):
        return v.reshape(1, -1)

    args = (
        x, communities.reshape(N, 1).astype(jnp.int32),
        p0['Wv'], p0['Wo'], p0['Wp'], row(p0['gamma']), row(p0['beta']),
        p1['Wv'], p1['Wo'], p1['Wp'], row(p1['gamma']), row(p1['beta']),
        p2['Wv'], p2['Wo'], p2['Wp'], row(p2['gamma']), row(p2['beta']),
        params['comm_emb'], ca['Wv'], ca['Wo'], row(ca['bv']), row(ca['bo']),
        params['pred_W1'], row(params['pred_b1']),
        params['pred_W2'], row(params['pred_b2']),
    )

    return pl.pallas_call(
        _gnn_kernel,
        out_shape=jax.ShapeDtypeStruct((N, D), jnp.float32),
        scratch_shapes=[pltpu.VMEM((N, D), jnp.float32)],
    )(*args)
